# baseline (speedup 1.0000x reference)
import functools

import jax
import jax.numpy as jnp
from jax import lax
from jax.experimental import pallas as pl
from jax.experimental.pallas import tpu as pltpu
from jax.experimental.pallas import tpu_sc as plsc

N_NODES = 10000
D = 128
DH = D // 2
N_EDGES = 320000

NC = 2
NS = 16
K = 80
NCHUNK = 256
E_PAD = NS * NCHUNK * K
NBUF = 8
LEAD = NBUF // 2
N_PAD = 10240
ROWS_PER_S = N_PAD // NS
ZROWS = 128

_mesh = plsc.VectorSubcoreMesh(core_axis_name="c", subcore_axis_name="s")


@functools.partial(
    pl.kernel,
    out_type=jax.ShapeDtypeStruct((NC, N_PAD, DH), jnp.float32),
    mesh=_mesh,
    scratch_types=(
        [
            pltpu.VMEM((NCHUNK, K), jnp.int32),
            pltpu.VMEM((NCHUNK, K), jnp.int32),
        ]
        + [pltpu.VMEM((K, DH), jnp.float32) for _ in range(NBUF)]
        + [
            pltpu.VMEM((ZROWS, DH), jnp.float32),
            pltpu.VMEM_SHARED((N_PAD, DH), jnp.float32),
        ]
        + [pltpu.SemaphoreType.DMA for _ in range(2 * NBUF)]
    ),
    compiler_params=pltpu.CompilerParams(use_tc_tiling_on_sc=False),
)
def _aggregate(src_hbm, dst_hbm, xl_hbm, xr_hbm, part_hbm, *scratch):
    src_v, dst_v = scratch[0], scratch[1]
    rows = scratch[2:2 + NBUF]
    zbuf, acc = scratch[2 + NBUF], scratch[3 + NBUF]
    gsem = scratch[4 + NBUF:4 + 2 * NBUF]
    ssem = scratch[4 + 2 * NBUF:4 + 3 * NBUF]
    c = lax.axis_index("c")
    s = lax.axis_index("s")

    pltpu.async_copy(src_hbm.at[s], src_v, gsem[0])
    pltpu.async_copy(dst_hbm.at[s], dst_v, gsem[1])

    def zrow(i, carry):
        def zcol(j, carry2):
            zbuf[i, pl.ds(j * 16, 16)] = jnp.zeros((16,), jnp.float32)
            return carry2
        return lax.fori_loop(0, DH // 16, zcol, carry)
    lax.fori_loop(0, ZROWS, zrow, 0)
    for r in range(ROWS_PER_S // ZROWS):
        pltpu.sync_copy(zbuf, acc.at[pl.ds(s * ROWS_PER_S + r * ZROWS, ZROWS)])

    pltpu.make_async_copy(src_hbm.at[s], src_v, gsem[0]).wait()
    pltpu.make_async_copy(dst_hbm.at[s], dst_v, gsem[1]).wait()
    plsc.subcore_barrier()

    def run(x_tab):
        def issue_g(i, b):
            pltpu.async_copy(x_tab.at[src_v.at[i]], rows[b], gsem[b])

        def wait_g(i, b):
            pltpu.make_async_copy(x_tab.at[src_v.at[i]], rows[b],
                                  gsem[b]).wait()

        def issue_s(i, b):
            pltpu.async_copy(rows[b], acc.at[dst_v.at[i]], ssem[b], add=True)

        def wait_s(i, b):
            pltpu.make_async_copy(rows[b], acc.at[dst_v.at[i]],
                                  ssem[b]).wait()

        for i in range(LEAD):
            issue_g(i, i % NBUF)

        for j in range(NBUF):
            wait_g(j, j)
            issue_s(j, j)
            if j >= LEAD:
                wait_s(j - LEAD, (j + LEAD) % NBUF)
            issue_g(j + LEAD, (j + LEAD) % NBUF)

        def body(g, carry):
            i0 = NBUF * g
            for j in range(NBUF):
                i = i0 + j
                wait_g(i, j)
                issue_s(i, j)
                wait_s(i - LEAD, (j + LEAD) % NBUF)
                issue_g(i + LEAD, (j + LEAD) % NBUF)
            return carry
        lax.fori_loop(1, NCHUNK // NBUF - 1, body, 0)

        i0 = NCHUNK - NBUF
        for j in range(NBUF):
            i = i0 + j
            wait_g(i, j)
            issue_s(i, j)
            wait_s(i - LEAD, (j + LEAD) % NBUF)
            if j < NBUF - LEAD:
                issue_g(i + LEAD, (j + LEAD) % NBUF)
        for j in range(LEAD):
            i = NCHUNK - LEAD + j
            wait_s(i, i % NBUF)

    pl.when(c == 0)(lambda: run(xl_hbm))
    pl.when(c == 1)(lambda: run(xr_hbm))

    plsc.subcore_barrier()

    pltpu.sync_copy(acc.at[pl.ds(s * ROWS_PER_S, ROWS_PER_S)],
                    part_hbm.at[c, pl.ds(s * ROWS_PER_S, ROWS_PER_S)])


BLK = 1000


def _mlp_body(x_ref, p0_ref, p1_ref, w1_ref, b1_ref, w2_ref, b2_ref, o_ref):
    msg = jnp.concatenate([p0_ref[...], p1_ref[...]], axis=1)
    h = x_ref[...] + msg
    h1 = jnp.dot(h, w1_ref[...], preferred_element_type=jnp.float32)
    h1 = jnp.maximum(h1 + b1_ref[...], 0.0)
    o_ref[...] = jnp.dot(h1, w2_ref[...],
                         preferred_element_type=jnp.float32) + b2_ref[...]


_mlp = pl.pallas_call(
    _mlp_body,
    out_shape=jax.ShapeDtypeStruct((N_NODES, D), jnp.float32),
    grid=(N_NODES // BLK,),
    in_specs=[
        pl.BlockSpec((BLK, D), lambda i: (i, 0)),
        pl.BlockSpec((BLK, DH), lambda i: (i, 0)),
        pl.BlockSpec((BLK, DH), lambda i: (i, 0)),
        pl.BlockSpec((D, D), lambda i: (0, 0)),
        pl.BlockSpec((1, D), lambda i: (0, 0)),
        pl.BlockSpec((D, D), lambda i: (0, 0)),
        pl.BlockSpec((1, D), lambda i: (0, 0)),
    ],
    out_specs=pl.BlockSpec((BLK, D), lambda i: (i, 0)),
)


def kernel(x, edge_index, degrees, W1, b1, W2, b2):
    npad = E_PAD - N_EDGES
    src = jnp.concatenate(
        [edge_index[0].astype(jnp.int32), jnp.zeros((npad,), jnp.int32)]
    ).reshape(NS, NCHUNK, K)
    dst = jnp.concatenate(
        [edge_index[1].astype(jnp.int32),
         jnp.full((npad,), N_NODES, jnp.int32)]
    ).reshape(NS, NCHUNK, K)
    part = _aggregate(src, dst, x[:, :DH], x[:, DH:])
    return _mlp(x, part[0], part[1], W1, b1.reshape(1, D), W2, b2.reshape(1, D))


# speedup vs baseline: 1.6540x; 1.6540x over previous
import functools

import jax
import jax.numpy as jnp
from jax import lax
from jax.experimental import pallas as pl
from jax.experimental.pallas import tpu as pltpu
from jax.experimental.pallas import tpu_sc as plsc

N_NODES = 10000
D = 128
DH = D // 2
N_EDGES = 320000

NC = 2
NS = 16
K = 80
NCHUNK = 252
E_PAD = NS * NCHUNK * K
NBUF = 6
LEAD = 4
SLACK = NBUF - LEAD
N_PAD = 10240
ROWS_PER_S = N_PAD // NS
ZROWS = 128

_mesh = plsc.VectorSubcoreMesh(core_axis_name="c", subcore_axis_name="s")


@functools.partial(
    pl.kernel,
    out_type=jax.ShapeDtypeStruct((NC, N_PAD, DH), jnp.float32),
    mesh=_mesh,
    scratch_types=(
        [
            pltpu.VMEM((NCHUNK, K), jnp.int32),
            pltpu.VMEM((NCHUNK, K), jnp.int32),
        ]
        + [pltpu.VMEM((K, DH), jnp.float32) for _ in range(NBUF)]
        + [
            pltpu.VMEM((ZROWS, DH), jnp.float32),
            pltpu.VMEM_SHARED((N_PAD, DH), jnp.float32),
        ]
        + [pltpu.SemaphoreType.DMA for _ in range(2 * NBUF)]
    ),
    compiler_params=pltpu.CompilerParams(use_tc_tiling_on_sc=False),
)
def _aggregate(src_hbm, dst_hbm, xl_hbm, xr_hbm, part_hbm, *scratch):
    src_v, dst_v = scratch[0], scratch[1]
    rows = scratch[2:2 + NBUF]
    zbuf, acc = scratch[2 + NBUF], scratch[3 + NBUF]
    gsem = scratch[4 + NBUF:4 + 2 * NBUF]
    ssem = scratch[4 + 2 * NBUF:4 + 3 * NBUF]
    c = lax.axis_index("c")
    s = lax.axis_index("s")

    pltpu.async_copy(src_hbm.at[s], src_v, gsem[0])
    pltpu.async_copy(dst_hbm.at[s], dst_v, gsem[1])

    def zrow(i, carry):
        def zcol(j, carry2):
            zbuf[i, pl.ds(j * 16, 16)] = jnp.zeros((16,), jnp.float32)
            return carry2
        return lax.fori_loop(0, DH // 16, zcol, carry)
    lax.fori_loop(0, ZROWS, zrow, 0)
    for r in range(ROWS_PER_S // ZROWS):
        pltpu.sync_copy(zbuf, acc.at[pl.ds(s * ROWS_PER_S + r * ZROWS, ZROWS)])

    pltpu.make_async_copy(src_hbm.at[s], src_v, gsem[0]).wait()
    pltpu.make_async_copy(dst_hbm.at[s], dst_v, gsem[1]).wait()
    plsc.subcore_barrier()

    def run(x_tab):
        def issue_g(i, b):
            pltpu.async_copy(x_tab.at[src_v.at[i]], rows[b], gsem[b])

        def wait_g(i, b):
            pltpu.make_async_copy(x_tab.at[src_v.at[i]], rows[b],
                                  gsem[b]).wait()

        def issue_s(i, b):
            pltpu.async_copy(rows[b], acc.at[dst_v.at[i]], ssem[b], add=True)

        def wait_s(i, b):
            pltpu.make_async_copy(rows[b], acc.at[dst_v.at[i]],
                                  ssem[b]).wait()

        for i in range(LEAD):
            issue_g(i, i % NBUF)

        for j in range(NBUF):
            wait_g(j, j)
            issue_s(j, j)
            if j >= SLACK:
                wait_s(j - SLACK, (j + LEAD) % NBUF)
            issue_g(j + LEAD, (j + LEAD) % NBUF)

        def body(g, carry):
            i0 = NBUF * g
            for j in range(NBUF):
                i = i0 + j
                wait_g(i, j)
                issue_s(i, j)
                wait_s(i - SLACK, (j + LEAD) % NBUF)
                issue_g(i + LEAD, (j + LEAD) % NBUF)
            return carry
        lax.fori_loop(1, NCHUNK // NBUF - 1, body, 0)

        i0 = NCHUNK - NBUF
        for j in range(NBUF):
            i = i0 + j
            wait_g(i, j)
            issue_s(i, j)
            wait_s(i - SLACK, (j + LEAD) % NBUF)
            if j < NBUF - LEAD:
                issue_g(i + LEAD, (j + LEAD) % NBUF)
        for j in range(SLACK):
            i = NCHUNK - SLACK + j
            wait_s(i, i % NBUF)

    pl.when(c == 0)(lambda: run(xl_hbm))
    pl.when(c == 1)(lambda: run(xr_hbm))

    plsc.subcore_barrier()

    pltpu.sync_copy(acc.at[pl.ds(s * ROWS_PER_S, ROWS_PER_S)],
                    part_hbm.at[c, pl.ds(s * ROWS_PER_S, ROWS_PER_S)])


BLK = 1000


def _mlp_body(x_ref, p0_ref, p1_ref, w1_ref, b1_ref, w2_ref, b2_ref, o_ref):
    msg = jnp.concatenate([p0_ref[...], p1_ref[...]], axis=1)
    h = x_ref[...] + msg
    h1 = jnp.dot(h, w1_ref[...], preferred_element_type=jnp.float32)
    h1 = jnp.maximum(h1 + b1_ref[...], 0.0)
    o_ref[...] = jnp.dot(h1, w2_ref[...],
                         preferred_element_type=jnp.float32) + b2_ref[...]


_mlp = pl.pallas_call(
    _mlp_body,
    out_shape=jax.ShapeDtypeStruct((N_NODES, D), jnp.float32),
    grid=(N_NODES // BLK,),
    in_specs=[
        pl.BlockSpec((BLK, D), lambda i: (i, 0)),
        pl.BlockSpec((BLK, DH), lambda i: (i, 0)),
        pl.BlockSpec((BLK, DH), lambda i: (i, 0)),
        pl.BlockSpec((D, D), lambda i: (0, 0)),
        pl.BlockSpec((1, D), lambda i: (0, 0)),
        pl.BlockSpec((D, D), lambda i: (0, 0)),
        pl.BlockSpec((1, D), lambda i: (0, 0)),
    ],
    out_specs=pl.BlockSpec((BLK, D), lambda i: (i, 0)),
)


def kernel(x, edge_index, degrees, W1, b1, W2, b2):
    npad = E_PAD - N_EDGES
    src = jnp.concatenate(
        [edge_index[0].astype(jnp.int32), jnp.zeros((npad,), jnp.int32)]
    ).reshape(NS, NCHUNK, K)
    dst = jnp.concatenate(
        [edge_index[1].astype(jnp.int32),
         jnp.full((npad,), N_NODES, jnp.int32)]
    ).reshape(NS, NCHUNK, K)
    part = _aggregate(src, dst, x[:, :DH], x[:, DH:])
    return _mlp(x, part[0], part[1], W1, b1.reshape(1, D), W2, b2.reshape(1, D))
